# baseline (speedup 1.0000x reference)
import functools

import jax
import jax.numpy as jnp
from jax import lax
from jax.experimental import pallas as pl
from jax.experimental.pallas import tpu as pltpu
from jax.experimental.pallas import tpu_sc as plsc

_info = plsc.get_sparse_core_info()
_NC, _NS = _info.num_cores, _info.num_subcores
_NW = _NC * _NS

_CHUNK = 128
_NBUF = 4


@functools.lru_cache(maxsize=None)
def _build_gather(n_tokens: int, embed: int, seq_l: int):
    b_per_w = n_tokens // _NW
    n_chunks = b_per_w // _CHUNK
    rounds = n_chunks // _NBUF
    n_b = n_tokens // seq_l
    tb_n = n_b // _CHUNK
    eh = embed // 8
    assert n_tokens % _NW == 0 and b_per_w % _CHUNK == 0
    assert n_chunks % _NBUF == 0 and rounds >= 2 and embed == 64
    mesh = plsc.VectorSubcoreMesh(core_axis_name="c", subcore_axis_name="s")

    @functools.partial(
        pl.kernel,
        mesh=mesh,
        out_type=jax.ShapeDtypeStruct((seq_l * eh, tb_n, 8 * _CHUNK),
                                      jnp.float32),
        scratch_types=[
            pltpu.VMEM((b_per_w,), jnp.int32),
            pltpu.VMEM((_NBUF, _CHUNK), jnp.int32),
            pltpu.VMEM((_NBUF, _CHUNK, 2 * embed), jnp.float32),
            pltpu.VMEM((_NBUF, eh, 1, 8 * _CHUNK), jnp.float32),
        ] + [pltpu.SemaphoreType.DMA] * (2 * _NBUF),
        compiler_params=pltpu.CompilerParams(
            use_tc_tiling_on_sc=False, needs_layout_passes=False),
    )
    def gather_kernel(idx_hbm, table_hbm, out_hbm, idx_all, gidx_v, rows_v,
                      buf_v, *sems):
        gsem, osem = sems[:_NBUF], sems[_NBUF:]
        wid = lax.axis_index("s") * _NC + lax.axis_index("c")
        base = wid * b_per_w
        base_blk = wid * n_chunks
        pltpu.sync_copy(idx_hbm.at[pl.ds(base, b_per_w)], idx_all)
        iota = lax.iota(jnp.int32, 16)

        def prep_gidx(i, b):
            ioff = i * _CHUNK

            def body(cg, carry):
                idxv = idx_all[pl.ds(ioff + cg * 16, 16)]
                gidx_v[b, pl.ds(cg * 16, 16)] = idxv >> 1
                return carry

            lax.fori_loop(0, _CHUNK // 16, body, 0)

        def start_gather(b):
            pltpu.async_copy(table_hbm.at[gidx_v.at[b]], rows_v.at[b],
                             gsem[b])

        def wait_gather(b):
            pltpu.make_async_copy(
                table_hbm.at[gidx_v.at[b]], rows_v.at[b], gsem[b]).wait()

        def permute(i, b):
            ioff = i * _CHUNK

            def body(cg, carry):
                idxv = idx_all[pl.ds(ioff + cg * 16, 16)]
                colb = (idxv & 1) * embed
                rowi = iota + cg * 16
                for e in range(embed):
                    vals = plsc.load_gather(rows_v.at[b], [rowi, colb + e])
                    buf_v[b, e // 8, 0,
                          pl.ds((e % 8) * _CHUNK + cg * 16, 16)] = vals
                return carry

            lax.fori_loop(0, _CHUNK // 16, body, 0)

        def start_store(i, b):
            blk = base_blk + i
            l8 = (blk // tb_n) * eh
            tb = blk % tb_n
            pltpu.async_copy(
                buf_v.at[b],
                out_hbm.at[pl.ds(l8, eh), pl.ds(tb, 1)], osem[b])

        def wait_store(b):
            pltpu.make_async_copy(
                buf_v.at[b], out_hbm.at[pl.ds(0, eh), pl.ds(0, 1)],
                osem[b]).wait()

        for b in range(_NBUF):
            prep_gidx(b, b)
            start_gather(b)
            if b >= 1:
                wait_gather(b - 1)
                permute(b - 1, b - 1)
                start_store(b - 1, b - 1)

        def round_body(r, carry):
            i0 = r * _NBUF
            for b in range(_NBUF):
                prev = (b - 1) % _NBUF
                wait_store(b)
                prep_gidx(i0 + b, b)
                start_gather(b)
                wait_gather(prev)
                permute(i0 + b - 1, prev)
                start_store(i0 + b - 1, prev)
            return carry

        lax.fori_loop(1, rounds, round_body, 0)

        lastb = _NBUF - 1
        wait_gather(lastb)
        permute(n_chunks - 1, lastb)
        start_store(n_chunks - 1, lastb)
        for b in range(_NBUF):
            wait_store(b)

    return gather_kernel


def kernel(inputtokens, table):
    b, l = inputtokens.shape
    v, e = table.shape
    flat = inputtokens.T.reshape(-1).astype(jnp.int32)
    table_p = table.reshape(v // 2, 2 * e)
    out = _build_gather(b * l, e, l)(flat, table_p)
    return (out.reshape(l, 8, b // 128, 8, 128)
               .transpose(2, 4, 0, 1, 3).reshape(b, l, e))


# speedup vs baseline: 1.3217x; 1.3217x over previous
import functools

import jax
import jax.numpy as jnp
from jax import lax
from jax.experimental import pallas as pl
from jax.experimental.pallas import tpu as pltpu
from jax.experimental.pallas import tpu_sc as plsc

_info = plsc.get_sparse_core_info()
_NC, _NS = _info.num_cores, _info.num_subcores
_NW = _NC * _NS

_CHUNK = 128
_NBUF = 4


@functools.lru_cache(maxsize=None)
def _build_gather(n_tokens: int, embed: int, seq_l: int):
    b_per_w = n_tokens // _NW
    n_chunks = b_per_w // _CHUNK
    rounds = n_chunks // _NBUF
    n_b = n_tokens // seq_l
    tb_n = n_b // _CHUNK
    eh = embed // 8
    assert n_tokens % _NW == 0 and b_per_w % _CHUNK == 0
    assert n_chunks % _NBUF == 0 and rounds >= 2 and embed == 64
    mesh = plsc.VectorSubcoreMesh(core_axis_name="c", subcore_axis_name="s")

    @functools.partial(
        pl.kernel,
        mesh=mesh,
        out_type=jax.ShapeDtypeStruct((seq_l * eh, tb_n, 8 * _CHUNK),
                                      jnp.float32),
        scratch_types=[
            pltpu.VMEM((b_per_w,), jnp.int32),
            pltpu.VMEM((_NBUF, _CHUNK), jnp.int32),
            pltpu.VMEM((_NBUF, _CHUNK, 2 * embed), jnp.float32),
            pltpu.VMEM((_NBUF, eh, 1, 8 * _CHUNK), jnp.float32),
        ] + [pltpu.SemaphoreType.DMA] * (2 * _NBUF),
        compiler_params=pltpu.CompilerParams(
            use_tc_tiling_on_sc=False, needs_layout_passes=False),
    )
    def gather_kernel(idx_hbm, table_hbm, out_hbm, idx_all, gidx_v, rows_v,
                      buf_v, *sems):
        gsem, osem = sems[:_NBUF], sems[_NBUF:]
        wid = lax.axis_index("s") * _NC + lax.axis_index("c")
        base = wid * b_per_w
        base_blk = wid * n_chunks
        pltpu.sync_copy(idx_hbm.at[pl.ds(base, b_per_w)], idx_all)
        iota = lax.iota(jnp.int32, 16)

        def prep_gidx(i, b):
            ioff = i * _CHUNK

            @plsc.parallel_loop(0, _CHUNK // 16, unroll=1)
            def _(cg):
                idxv = idx_all[pl.ds(ioff + cg * 16, 16)]
                gidx_v[b, pl.ds(cg * 16, 16)] = idxv >> 1

        def start_gather(b):
            pltpu.async_copy(table_hbm.at[gidx_v.at[b]], rows_v.at[b],
                             gsem[b])

        def wait_gather(b):
            pltpu.make_async_copy(
                table_hbm.at[gidx_v.at[b]], rows_v.at[b], gsem[b]).wait()

        def permute(i, b):
            ioff = i * _CHUNK

            @plsc.parallel_loop(0, _CHUNK // 16, unroll=1)
            def _(cg):
                idxv = idx_all[pl.ds(ioff + cg * 16, 16)]
                colb = (idxv & 1) * embed
                rowi = iota + cg * 16
                for e in range(embed):
                    vals = plsc.load_gather(rows_v.at[b], [rowi, colb + e])
                    buf_v[b, e // 8, 0,
                          pl.ds((e % 8) * _CHUNK + cg * 16, 16)] = vals

        def start_store(i, b):
            blk = base_blk + i
            l8 = (blk // tb_n) * eh
            tb = blk % tb_n
            pltpu.async_copy(
                buf_v.at[b],
                out_hbm.at[pl.ds(l8, eh), pl.ds(tb, 1)], osem[b])

        def wait_store(b):
            pltpu.make_async_copy(
                buf_v.at[b], out_hbm.at[pl.ds(0, eh), pl.ds(0, 1)],
                osem[b]).wait()

        for b in range(_NBUF):
            prep_gidx(b, b)
            start_gather(b)
            if b >= 1:
                wait_gather(b - 1)
                permute(b - 1, b - 1)
                start_store(b - 1, b - 1)

        def round_body(r, carry):
            i0 = r * _NBUF
            for b in range(_NBUF):
                prev = (b - 1) % _NBUF
                wait_store(b)
                prep_gidx(i0 + b, b)
                start_gather(b)
                wait_gather(prev)
                permute(i0 + b - 1, prev)
                start_store(i0 + b - 1, prev)
            return carry

        lax.fori_loop(1, rounds, round_body, 0)

        lastb = _NBUF - 1
        wait_gather(lastb)
        permute(n_chunks - 1, lastb)
        start_store(n_chunks - 1, lastb)
        for b in range(_NBUF):
            wait_store(b)

    return gather_kernel


def kernel(inputtokens, table):
    b, l = inputtokens.shape
    v, e = table.shape
    flat = inputtokens.T.reshape(-1).astype(jnp.int32)
    table_p = table.reshape(v // 2, 2 * e)
    out = _build_gather(b * l, e, l)(flat, table_p)
    return (out.reshape(l, 8, b // 128, 8, 128)
               .transpose(2, 4, 0, 1, 3).reshape(b, l, e))
